# baseline (speedup 1.0000x reference)
import functools

import jax
import jax.numpy as jnp
from jax import lax
from jax.experimental import pallas as pl
from jax.experimental.pallas import tpu as pltpu
from jax.experimental.pallas import tpu_sc as plsc

B = 16384
D = 32
NC = 2
NS = 16
NW = NC * NS
BPW = B // NW
CH = 128
NCH = BPW // CH
L = 16


def _bpr_body(user_hbm, item_i_hbm, item_j_hbm, utab_hbm, itab_hbm,
              out_i_hbm, out_j_hbm,
              uidx_v, iidx_v, jidx_v, u_v, vi_v, vj_v, oi_v, oj_v, sem):
    wid = lax.axis_index("s") * NC + lax.axis_index("c")
    base = wid * BPW

    pltpu.sync_copy(user_hbm.at[pl.ds(base, BPW)], uidx_v)
    pltpu.sync_copy(item_i_hbm.at[pl.ds(base, BPW)], iidx_v)
    pltpu.sync_copy(item_j_hbm.at[pl.ds(base, BPW)], jidx_v)

    def chunk(c, _):
        cp_u = pltpu.async_copy(utab_hbm.at[uidx_v.at[pl.ds(c * CH, CH)]], u_v, sem)
        cp_i = pltpu.async_copy(itab_hbm.at[iidx_v.at[pl.ds(c * CH, CH)]], vi_v, sem)
        cp_j = pltpu.async_copy(itab_hbm.at[jidx_v.at[pl.ds(c * CH, CH)]], vj_v, sem)
        cp_u.wait()
        cp_i.wait()
        cp_j.wait()

        def group(g, _):
            rows = g * L + lax.iota(jnp.int32, L)
            acc_i = jnp.zeros((L,), jnp.float32)
            acc_j = jnp.zeros((L,), jnp.float32)
            for d in range(D):
                cols = jnp.full((L,), d, jnp.int32)
                uc = plsc.load_gather(u_v, [rows, cols])
                ic = plsc.load_gather(vi_v, [rows, cols])
                jc = plsc.load_gather(vj_v, [rows, cols])
                acc_i = acc_i + uc * ic
                acc_j = acc_j + uc * jc
            out = c * CH + g * L
            oi_v[pl.ds(out, L)] = acc_i
            oj_v[pl.ds(out, L)] = acc_j
            return 0

        lax.fori_loop(0, CH // L, group, 0)
        return 0

    lax.fori_loop(0, NCH, chunk, 0)

    pltpu.sync_copy(oi_v, out_i_hbm.at[pl.ds(base, BPW)])
    pltpu.sync_copy(oj_v, out_j_hbm.at[pl.ds(base, BPW)])


@jax.jit
def _bpr(user, item_i, item_j, embed_user_weight, embed_item_weight):
    mesh = plsc.VectorSubcoreMesh(core_axis_name="c", subcore_axis_name="s")
    run = pl.kernel(
        _bpr_body,
        out_type=(jax.ShapeDtypeStruct((B,), jnp.float32),
                  jax.ShapeDtypeStruct((B,), jnp.float32)),
        mesh=mesh,
        compiler_params=pltpu.CompilerParams(
            needs_layout_passes=False, use_tc_tiling_on_sc=False),
        scratch_types=[
            pltpu.VMEM((BPW,), jnp.int32),
            pltpu.VMEM((BPW,), jnp.int32),
            pltpu.VMEM((BPW,), jnp.int32),
            pltpu.VMEM((CH, D), jnp.float32),
            pltpu.VMEM((CH, D), jnp.float32),
            pltpu.VMEM((CH, D), jnp.float32),
            pltpu.VMEM((BPW,), jnp.float32),
            pltpu.VMEM((BPW,), jnp.float32),
            pltpu.SemaphoreType.DMA,
        ],
    )
    return run(user, item_i, item_j, embed_user_weight, embed_item_weight)


def kernel(user, item_i, item_j, embed_user_weight, embed_item_weight):
    return _bpr(user.astype(jnp.int32), item_i.astype(jnp.int32),
                item_j.astype(jnp.int32), embed_user_weight, embed_item_weight)
